# baseline (speedup 1.0000x reference)
import functools

import jax
import jax.numpy as jnp
from jax import lax
from jax.experimental import pallas as pl
from jax.experimental.pallas import tpu as pltpu
from jax.experimental.pallas import tpu_sc as plsc

NC, NS = 2, 16
NW = NC * NS
LB = 128
KCH = 16
LANES = 16


def _mesh():
  return plsc.VectorSubcoreMesh(core_axis_name="c", subcore_axis_name="s")


def _split_rows(nrows):
  bpt = (nrows // NW) // (2 * KCH) * (2 * KCH)
  tail_units = (nrows - NW * bpt) // 8
  return bpt, tail_units


def _tail_geometry(wid, tail_units):
  base_u = tail_units // NW
  extra = tail_units % NW
  u = base_u + jnp.where(wid < extra, 1, 0)
  first = wid * base_u + jnp.minimum(wid, extra)
  max_u = -(-tail_units // NW) if tail_units else 0
  return u, first, max_u


def _zero_acc_slice(bnc, acc_sh, s, n_sl):
  def _z(i, carry):
    bnc[pl.ds(i * LANES, LANES)] = jnp.zeros((LANES,), jnp.float32)
    return carry
  lax.fori_loop(0, n_sl // LANES, _z, None)
  pltpu.sync_copy(bnc, acc_sh.at[pl.ds(s * n_sl, n_sl)])


def _write_out(bnc, acc_sh, out_h, c, s, n_sl, npad):
  pltpu.sync_copy(acc_sh.at[pl.ds(s * n_sl, n_sl)], bnc)
  pltpu.sync_copy(bnc, out_h.at[pl.ds(c * npad + s * n_sl, n_sl)])


def _deg_pass(nrows, npad, n_sl):
  bpt, tail_units = _split_rows(nrows)
  scratch = [
      pltpu.VMEM((2 * KCH, LB), jnp.int32),
      pltpu.VMEM((KCH * LB,), jnp.float32),
      pltpu.VMEM((n_sl,), jnp.float32),
      pltpu.SemaphoreType.DMA,
      pltpu.VMEM_SHARED((npad,), jnp.float32),
  ]

  def body(dst2, out_h, didx, vals, bnc, ssem, acc_sh):
    c = lax.axis_index("c")
    s = lax.axis_index("s")
    wid = c * NS + s
    _zero_acc_slice(bnc, acc_sh, s, n_sl)

    def _o(i, carry):
      vals[pl.ds(i * LANES, LANES)] = jnp.full((LANES,), 1.0, jnp.float32)
      return carry
    lax.fori_loop(0, (KCH * LB) // LANES, _o, None)
    plsc.subcore_barrier()

    row0 = wid * bpt

    def scatter_rows(nr):
      sds = [
          pltpu.async_copy(vals.at[pl.ds(j * LB, LB)],
                           acc_sh.at[didx.at[j]], ssem, add=True)
          for j in range(nr)
      ]
      for d in sds:
        d.wait()

    def chunk(ch, carry):
      pltpu.sync_copy(dst2.at[pl.ds(row0 + ch * KCH, KCH)],
                      didx.at[pl.ds(0, KCH)])
      scatter_rows(KCH)
      return carry

    lax.fori_loop(0, bpt // KCH, chunk, None)

    if tail_units:
      u, first, max_u = _tail_geometry(wid, tail_units)
      for uu in range(1, max_u + 1):
        @pl.when(u == uu)
        def _(uu=uu):
          nr = 8 * uu
          pltpu.sync_copy(dst2.at[pl.ds(NW * bpt + 8 * first, nr)],
                          didx.at[pl.ds(0, nr)])
          scatter_rows(nr)

    plsc.subcore_barrier()
    _write_out(bnc, acc_sh, out_h, c, s, n_sl, npad)

  return pl.kernel(
      body,
      out_type=jax.ShapeDtypeStruct((NC * npad,), jnp.float32),
      mesh=_mesh(),
      scratch_types=scratch,
  )


def _conv_pass(nrows, npad, n_sl):
  bpt, tail_units = _split_rows(nrows)
  scratch = [
      pltpu.VMEM((npad,), jnp.float32),
      pltpu.VMEM((2 * KCH, LB), jnp.int32),
      pltpu.VMEM((n_sl,), jnp.float32),
      pltpu.VMEM_SHARED((npad,), jnp.float32),
  ]
  for _ in range(2):
    scratch += [
        pltpu.VMEM((KCH, LB), jnp.int32),
        pltpu.VMEM((KCH, LB), jnp.float32),
        pltpu.SemaphoreType.DMA,
    ]

  def body(src2, dst2, tbl_h, out_h, tblv, sidx, bnc, acc_sh,
           didx0, vals0, ssem0, didx1, vals1, ssem1):
    c = lax.axis_index("c")
    s = lax.axis_index("s")
    wid = c * NS + s
    _zero_acc_slice(bnc, acc_sh, s, n_sl)
    pltpu.sync_copy(tbl_h, tblv)
    plsc.subcore_barrier()

    row0 = wid * bpt
    sets = ((didx0, vals0, ssem0), (didx1, vals1, ssem1))

    def drains(didx, vals, ssem, nr):
      return [
          pltpu.make_async_copy(vals.at[j], acc_sh.at[didx.at[j]], ssem)
          for j in range(nr)
      ]

    def process(ch, nr, soff, didx, vals, ssem):
      pltpu.sync_copy(src2.at[pl.ds(ch, nr)], sidx.at[pl.ds(soff, nr)])
      pltpu.sync_copy(dst2.at[pl.ds(ch, nr)], didx.at[pl.ds(0, nr)])
      for j in range(nr):
        for b in range(LB // LANES):
          iv = sidx[soff + j, pl.ds(b * LANES, LANES)]
          vals[j, pl.ds(b * LANES, LANES)] = plsc.load_gather(tblv, [iv])
        pltpu.async_copy(vals.at[j], acc_sh.at[didx.at[j]], ssem, add=True)

    def pair(t, carry):
      base = row0 + t * (2 * KCH)
      for half, (didx, vals, ssem) in enumerate(sets):
        @pl.when(t > 0)
        def _(didx=didx, vals=vals, ssem=ssem):
          for d in drains(didx, vals, ssem, KCH):
            d.wait()
        process(base + half * KCH, KCH, half * KCH, didx, vals, ssem)
      return carry

    lax.fori_loop(0, bpt // (2 * KCH), pair, None)
    for d in drains(*sets[0], KCH):
      d.wait()

    if tail_units:
      u, first, max_u = _tail_geometry(wid, tail_units)
      for uu in range(1, max_u + 1):
        @pl.when(u == uu)
        def _(uu=uu):
          nr = 8 * uu
          process(NW * bpt + 8 * first, nr, 0, *sets[0])
          for d in drains(*sets[0], nr):
            d.wait()

    for d in drains(*sets[1], KCH):
      d.wait()
    plsc.subcore_barrier()
    _write_out(bnc, acc_sh, out_h, c, s, n_sl, npad)

  return pl.kernel(
      body,
      out_type=jax.ShapeDtypeStruct((NC * npad,), jnp.float32),
      mesh=_mesh(),
      scratch_types=scratch,
      compiler_params=pltpu.CompilerParams(needs_layout_passes=False),
  )


def _tc_prep(rows):
  def body(degp, xp, dis, p):
    deg = degp[0] + degp[1] + 1.0
    d = lax.rsqrt(deg)
    d = d * (1.5 - 0.5 * deg * d * d)
    d = d * (1.5 - 0.5 * deg * d * d)
    dis[...] = d
    p[...] = d * xp[...]

  return pl.pallas_call(
      body,
      out_shape=[jax.ShapeDtypeStruct((rows, 128), jnp.float32)] * 2,
  )


def _tc_act(rows, width):
  def body(sp, dis, xp, w1, b1, w2, q):
    d = dis[...]
    s1 = d * (sp[0] + sp[1] + d * xp[...])
    acc = jnp.zeros_like(s1)
    for k in range(width):
      r = jnp.maximum(s1 * w1[k] + b1[k], 0.0)
      r = r.astype(jnp.bfloat16).astype(jnp.float32)
      w2k = w2[k].astype(jnp.bfloat16).astype(jnp.float32)
      acc = acc + r * w2k
    q[...] = d * acc

  smem = pl.BlockSpec(memory_space=pltpu.SMEM)
  return pl.pallas_call(
      body,
      in_specs=[pl.BlockSpec((NC, rows, 128), lambda: (0, 0, 0)),
                pl.BlockSpec((rows, 128), lambda: (0, 0)),
                pl.BlockSpec((rows, 128), lambda: (0, 0)),
                smem, smem, smem],
      out_shape=jax.ShapeDtypeStruct((rows, 128), jnp.float32),
  )


def _tc_fin(rows, n):
  def body(sp, dis, q, b2, out):
    d = dis[...]
    v = d * (sp[0] + sp[1]) + d * q[...] + b2[0]
    rid = lax.broadcasted_iota(jnp.int32, (rows, 128), 0)
    cid = lax.broadcasted_iota(jnp.int32, (rows, 128), 1)
    v = jnp.where(rid * 128 + cid < n, v, -jnp.inf)
    out[...] = jnp.max(v).reshape(1, 1)

  smem = pl.BlockSpec(memory_space=pltpu.SMEM)
  return pl.pallas_call(
      body,
      in_specs=[pl.BlockSpec((NC, rows, 128), lambda: (0, 0, 0)),
                pl.BlockSpec((rows, 128), lambda: (0, 0)),
                pl.BlockSpec((rows, 128), lambda: (0, 0)),
                smem],
      out_shape=jax.ShapeDtypeStruct((1, 1), jnp.float32),
  )


def kernel(x, edge_index, W1, b1, W2, b2):
  n = x.shape[0]
  e = edge_index.shape[1]
  width = W1.shape[1]

  n_sl = -(-(n + 1) // (NS * 8)) * 8
  npad = NS * n_sl
  rows = npad // 128
  nrows = e // LB

  src2 = edge_index[0].reshape(nrows, LB)
  dst2 = edge_index[1].reshape(nrows, LB)
  xp = jnp.pad(x[:, 0], (0, npad - n)).reshape(rows, 128)

  conv = _conv_pass(nrows, npad, n_sl)

  degp = _deg_pass(nrows, npad, n_sl)(dst2)
  dis, p = _tc_prep(rows)(degp.reshape(NC, rows, 128), xp)
  sp1 = conv(src2, dst2, p.reshape(npad))
  q = _tc_act(rows, width)(sp1.reshape(NC, rows, 128), dis, xp,
                           W1.reshape(width), b1, W2.reshape(width))
  sp2 = conv(src2, dst2, q.reshape(npad))
  return _tc_fin(rows, n)(sp2.reshape(NC, rows, 128), dis, q, b2)


# speedup vs baseline: 1.0128x; 1.0128x over previous
import functools

import jax
import jax.numpy as jnp
from jax import lax
from jax.experimental import pallas as pl
from jax.experimental.pallas import tpu as pltpu
from jax.experimental.pallas import tpu_sc as plsc

NC, NS = 2, 16
NW = NC * NS
LB = 128
KCH = 16
LANES = 16


def _mesh():
  return plsc.VectorSubcoreMesh(core_axis_name="c", subcore_axis_name="s")


def _split_rows(nrows):
  bpt = (nrows // NW) // (2 * KCH) * (2 * KCH)
  tail_units = (nrows - NW * bpt) // 8
  return bpt, tail_units


def _tail_geometry(wid, tail_units):
  base_u = tail_units // NW
  extra = tail_units % NW
  u = base_u + jnp.where(wid < extra, 1, 0)
  first = wid * base_u + jnp.minimum(wid, extra)
  max_u = -(-tail_units // NW) if tail_units else 0
  return u, first, max_u


def _zero_acc_slice(bnc, acc_sh, s, n_sl):
  def _z(i, carry):
    bnc[pl.ds(i * LANES, LANES)] = jnp.zeros((LANES,), jnp.float32)
    return carry
  lax.fori_loop(0, n_sl // LANES, _z, None)
  pltpu.sync_copy(bnc, acc_sh.at[pl.ds(s * n_sl, n_sl)])


def _write_out(bnc, acc_sh, out_h, c, s, n_sl, npad):
  pltpu.sync_copy(acc_sh.at[pl.ds(s * n_sl, n_sl)], bnc)
  pltpu.sync_copy(bnc, out_h.at[pl.ds(c * npad + s * n_sl, n_sl)])


def _deg_pass(nrows, npad, n_sl):
  bpt, tail_units = _split_rows(nrows)
  scratch = [
      pltpu.VMEM((2 * KCH, LB), jnp.int32),
      pltpu.VMEM((KCH * LB,), jnp.float32),
      pltpu.VMEM((n_sl,), jnp.float32),
      pltpu.SemaphoreType.DMA,
      pltpu.VMEM_SHARED((npad,), jnp.float32),
  ]

  def body(dst2, out_h, didx, vals, bnc, ssem, acc_sh):
    c = lax.axis_index("c")
    s = lax.axis_index("s")
    wid = c * NS + s
    _zero_acc_slice(bnc, acc_sh, s, n_sl)

    def _o(i, carry):
      vals[pl.ds(i * LANES, LANES)] = jnp.full((LANES,), 1.0, jnp.float32)
      return carry
    lax.fori_loop(0, (KCH * LB) // LANES, _o, None)
    plsc.subcore_barrier()

    row0 = wid * bpt

    def scatter_rows(nr):
      sds = [
          pltpu.async_copy(vals.at[pl.ds(j * LB, LB)],
                           acc_sh.at[didx.at[j]], ssem, add=True)
          for j in range(nr)
      ]
      for d in sds:
        d.wait()

    def chunk(ch, carry):
      pltpu.sync_copy(dst2.at[pl.ds(row0 + ch * KCH, KCH)],
                      didx.at[pl.ds(0, KCH)])
      scatter_rows(KCH)
      return carry

    lax.fori_loop(0, bpt // KCH, chunk, None)

    if tail_units:
      u, first, max_u = _tail_geometry(wid, tail_units)
      for uu in range(1, max_u + 1):
        @pl.when(u == uu)
        def _(uu=uu):
          nr = 8 * uu
          pltpu.sync_copy(dst2.at[pl.ds(NW * bpt + 8 * first, nr)],
                          didx.at[pl.ds(0, nr)])
          scatter_rows(nr)

    plsc.subcore_barrier()
    _write_out(bnc, acc_sh, out_h, c, s, n_sl, npad)

  return pl.kernel(
      body,
      out_type=jax.ShapeDtypeStruct((NC * npad,), jnp.float32),
      mesh=_mesh(),
      scratch_types=scratch,
  )


def _conv_pass(nrows, npad, n_sl):
  bpt, tail_units = _split_rows(nrows)
  scratch = [
      pltpu.VMEM((npad,), jnp.float32),
      pltpu.VMEM((2 * KCH, LB), jnp.int32),
      pltpu.VMEM((n_sl,), jnp.float32),
      pltpu.VMEM_SHARED((npad,), jnp.float32),
  ]
  for _ in range(2):
    scratch += [
        pltpu.VMEM((KCH, LB), jnp.int32),
        pltpu.VMEM((KCH, LB), jnp.float32),
        pltpu.SemaphoreType.DMA,
    ]

  def body(src2, dst2, tbl_h, out_h, tblv, sidx, bnc, acc_sh,
           didx0, vals0, ssem0, didx1, vals1, ssem1):
    c = lax.axis_index("c")
    s = lax.axis_index("s")
    wid = c * NS + s
    _zero_acc_slice(bnc, acc_sh, s, n_sl)
    pltpu.sync_copy(tbl_h, tblv)
    plsc.subcore_barrier()

    row0 = wid * bpt
    sets = ((didx0, vals0, ssem0), (didx1, vals1, ssem1))

    def drains(didx, vals, ssem, nr):
      return [
          pltpu.make_async_copy(vals.at[j], acc_sh.at[didx.at[j]], ssem)
          for j in range(nr)
      ]

    def process(ch, nr, soff, didx, vals, ssem):
      pltpu.sync_copy(src2.at[pl.ds(ch, nr)], sidx.at[pl.ds(soff, nr)])
      pltpu.sync_copy(dst2.at[pl.ds(ch, nr)], didx.at[pl.ds(0, nr)])
      for j in range(nr):
        for b in range(LB // LANES):
          iv = sidx[soff + j, pl.ds(b * LANES, LANES)]
          vals[j, pl.ds(b * LANES, LANES)] = plsc.load_gather(tblv, [iv])
      for j in range(nr):
        pltpu.async_copy(vals.at[j], acc_sh.at[didx.at[j]], ssem, add=True)

    def pair(t, carry):
      base = row0 + t * (2 * KCH)
      for half, (didx, vals, ssem) in enumerate(sets):
        @pl.when(t > 0)
        def _(didx=didx, vals=vals, ssem=ssem):
          for d in drains(didx, vals, ssem, KCH):
            d.wait()
        process(base + half * KCH, KCH, half * KCH, didx, vals, ssem)
      return carry

    lax.fori_loop(0, bpt // (2 * KCH), pair, None)
    for d in drains(*sets[0], KCH):
      d.wait()

    if tail_units:
      u, first, max_u = _tail_geometry(wid, tail_units)
      for uu in range(1, max_u + 1):
        @pl.when(u == uu)
        def _(uu=uu):
          nr = 8 * uu
          process(NW * bpt + 8 * first, nr, 0, *sets[0])
          for d in drains(*sets[0], nr):
            d.wait()

    for d in drains(*sets[1], KCH):
      d.wait()
    plsc.subcore_barrier()
    _write_out(bnc, acc_sh, out_h, c, s, n_sl, npad)

  return pl.kernel(
      body,
      out_type=jax.ShapeDtypeStruct((NC * npad,), jnp.float32),
      mesh=_mesh(),
      scratch_types=scratch,
      compiler_params=pltpu.CompilerParams(needs_layout_passes=False),
  )


def _tc_prep(rows):
  def body(degp, xp, dis, p):
    deg = degp[0] + degp[1] + 1.0
    d = lax.rsqrt(deg)
    d = d * (1.5 - 0.5 * deg * d * d)
    d = d * (1.5 - 0.5 * deg * d * d)
    dis[...] = d
    p[...] = d * xp[...]

  return pl.pallas_call(
      body,
      out_shape=[jax.ShapeDtypeStruct((rows, 128), jnp.float32)] * 2,
  )


def _tc_act(rows, width):
  def body(sp, dis, xp, w1, b1, w2, q):
    d = dis[...]
    s1 = d * (sp[0] + sp[1] + d * xp[...])
    acc = jnp.zeros_like(s1)
    for k in range(width):
      r = jnp.maximum(s1 * w1[k] + b1[k], 0.0)
      r = r.astype(jnp.bfloat16).astype(jnp.float32)
      w2k = w2[k].astype(jnp.bfloat16).astype(jnp.float32)
      acc = acc + r * w2k
    q[...] = d * acc

  smem = pl.BlockSpec(memory_space=pltpu.SMEM)
  return pl.pallas_call(
      body,
      in_specs=[pl.BlockSpec((NC, rows, 128), lambda: (0, 0, 0)),
                pl.BlockSpec((rows, 128), lambda: (0, 0)),
                pl.BlockSpec((rows, 128), lambda: (0, 0)),
                smem, smem, smem],
      out_shape=jax.ShapeDtypeStruct((rows, 128), jnp.float32),
  )


def _tc_fin(rows, n):
  def body(sp, dis, q, b2, out):
    d = dis[...]
    v = d * (sp[0] + sp[1]) + d * q[...] + b2[0]
    rid = lax.broadcasted_iota(jnp.int32, (rows, 128), 0)
    cid = lax.broadcasted_iota(jnp.int32, (rows, 128), 1)
    v = jnp.where(rid * 128 + cid < n, v, -jnp.inf)
    out[...] = jnp.max(v).reshape(1, 1)

  smem = pl.BlockSpec(memory_space=pltpu.SMEM)
  return pl.pallas_call(
      body,
      in_specs=[pl.BlockSpec((NC, rows, 128), lambda: (0, 0, 0)),
                pl.BlockSpec((rows, 128), lambda: (0, 0)),
                pl.BlockSpec((rows, 128), lambda: (0, 0)),
                smem],
      out_shape=jax.ShapeDtypeStruct((1, 1), jnp.float32),
  )


def kernel(x, edge_index, W1, b1, W2, b2):
  n = x.shape[0]
  e = edge_index.shape[1]
  width = W1.shape[1]

  n_sl = -(-(n + 1) // (NS * 8)) * 8
  npad = NS * n_sl
  rows = npad // 128
  nrows = e // LB

  src2 = edge_index[0].reshape(nrows, LB)
  dst2 = edge_index[1].reshape(nrows, LB)
  xp = jnp.pad(x[:, 0], (0, npad - n)).reshape(rows, 128)

  conv = _conv_pass(nrows, npad, n_sl)

  degp = _deg_pass(nrows, npad, n_sl)(dst2)
  dis, p = _tc_prep(rows)(degp.reshape(NC, rows, 128), xp)
  sp1 = conv(src2, dst2, p.reshape(npad))
  q = _tc_act(rows, width)(sp1.reshape(NC, rows, 128), dis, xp,
                           W1.reshape(width), b1, W2.reshape(width))
  sp2 = conv(src2, dst2, q.reshape(npad))
  return _tc_fin(rows, n)(sp2.reshape(NC, rows, 128), dis, q, b2)


# speedup vs baseline: 1.0260x; 1.0131x over previous
import functools

import jax
import jax.numpy as jnp
from jax import lax
from jax.experimental import pallas as pl
from jax.experimental.pallas import tpu as pltpu
from jax.experimental.pallas import tpu_sc as plsc

NC, NS = 2, 16
NW = NC * NS
LB = 128
KCH = 16
LANES = 16


def _mesh():
  return plsc.VectorSubcoreMesh(core_axis_name="c", subcore_axis_name="s")


def _split_rows(nrows):
  bpt = (nrows // NW) // (2 * KCH) * (2 * KCH)
  tail_units = (nrows - NW * bpt) // 8
  return bpt, tail_units


def _tail_geometry(wid, tail_units):
  base_u = tail_units // NW
  extra = tail_units % NW
  u = base_u + jnp.where(wid < extra, 1, 0)
  first = wid * base_u + jnp.minimum(wid, extra)
  max_u = -(-tail_units // NW) if tail_units else 0
  return u, first, max_u


def _zero_acc_slice(bnc, acc_sh, s, n_sl):
  def _z(i, carry):
    bnc[pl.ds(i * LANES, LANES)] = jnp.zeros((LANES,), jnp.float32)
    return carry
  lax.fori_loop(0, n_sl // LANES, _z, None)
  pltpu.sync_copy(bnc, acc_sh.at[pl.ds(s * n_sl, n_sl)])


def _write_out(bnc, acc_sh, out_h, c, s, n_sl, npad):
  pltpu.sync_copy(acc_sh.at[pl.ds(s * n_sl, n_sl)], bnc)
  pltpu.sync_copy(bnc, out_h.at[pl.ds(c * npad + s * n_sl, n_sl)])


def _deg_pass(nrows, npad, n_sl):
  bpt, tail_units = _split_rows(nrows)
  scratch = [
      pltpu.VMEM((2 * KCH, LB), jnp.int32),
      pltpu.VMEM((KCH * LB,), jnp.float32),
      pltpu.VMEM((n_sl,), jnp.float32),
      pltpu.SemaphoreType.DMA,
      pltpu.VMEM_SHARED((npad,), jnp.float32),
  ]

  def body(ei3, out_h, didx, vals, bnc, ssem, acc_sh):
    c = lax.axis_index("c")
    s = lax.axis_index("s")
    wid = c * NS + s
    _zero_acc_slice(bnc, acc_sh, s, n_sl)

    def _o(i, carry):
      vals[pl.ds(i * LANES, LANES)] = jnp.full((LANES,), 1.0, jnp.float32)
      return carry
    lax.fori_loop(0, (KCH * LB) // LANES, _o, None)
    plsc.subcore_barrier()

    row0 = wid * bpt

    def scatter_rows(nr):
      sds = [
          pltpu.async_copy(vals.at[pl.ds(j * LB, LB)],
                           acc_sh.at[didx.at[j]], ssem, add=True)
          for j in range(nr)
      ]
      for d in sds:
        d.wait()

    def chunk(ch, carry):
      pltpu.sync_copy(ei3.at[1, pl.ds(row0 + ch * KCH, KCH)],
                      didx.at[pl.ds(0, KCH)])
      scatter_rows(KCH)
      return carry

    lax.fori_loop(0, bpt // KCH, chunk, None)

    if tail_units:
      u, first, max_u = _tail_geometry(wid, tail_units)
      for uu in range(1, max_u + 1):
        @pl.when(u == uu)
        def _(uu=uu):
          nr = 8 * uu
          pltpu.sync_copy(ei3.at[1, pl.ds(NW * bpt + 8 * first, nr)],
                          didx.at[pl.ds(0, nr)])
          scatter_rows(nr)

    plsc.subcore_barrier()
    _write_out(bnc, acc_sh, out_h, c, s, n_sl, npad)

  return pl.kernel(
      body,
      out_type=jax.ShapeDtypeStruct((NC * npad,), jnp.float32),
      mesh=_mesh(),
      scratch_types=scratch,
  )


def _conv_pass(nrows, npad, n_sl):
  bpt, tail_units = _split_rows(nrows)
  scratch = [
      pltpu.VMEM((npad,), jnp.float32),
      pltpu.VMEM((2 * KCH, LB), jnp.int32),
      pltpu.VMEM((n_sl,), jnp.float32),
      pltpu.VMEM_SHARED((npad,), jnp.float32),
  ]
  for _ in range(2):
    scratch += [
        pltpu.VMEM((KCH, LB), jnp.int32),
        pltpu.VMEM((KCH, LB), jnp.float32),
        pltpu.SemaphoreType.DMA,
    ]

  def body(ei3, tbl_h, out_h, tblv, sidx, bnc, acc_sh,
           didx0, vals0, ssem0, didx1, vals1, ssem1):
    c = lax.axis_index("c")
    s = lax.axis_index("s")
    wid = c * NS + s
    _zero_acc_slice(bnc, acc_sh, s, n_sl)
    pltpu.sync_copy(tbl_h, tblv)
    plsc.subcore_barrier()

    row0 = wid * bpt
    sets = ((didx0, vals0, ssem0), (didx1, vals1, ssem1))

    def drains(didx, vals, ssem, nr):
      return [
          pltpu.make_async_copy(vals.at[j], acc_sh.at[didx.at[j]], ssem)
          for j in range(nr)
      ]

    def process(ch, nr, soff, didx, vals, ssem):
      pltpu.sync_copy(ei3.at[0, pl.ds(ch, nr)], sidx.at[pl.ds(soff, nr)])
      pltpu.sync_copy(ei3.at[1, pl.ds(ch, nr)], didx.at[pl.ds(0, nr)])
      for j in range(nr):
        for b in range(LB // LANES):
          iv = sidx[soff + j, pl.ds(b * LANES, LANES)]
          vals[j, pl.ds(b * LANES, LANES)] = plsc.load_gather(tblv, [iv])
      for j in range(nr):
        pltpu.async_copy(vals.at[j], acc_sh.at[didx.at[j]], ssem, add=True)

    def pair(t, carry):
      base = row0 + t * (2 * KCH)
      for half, (didx, vals, ssem) in enumerate(sets):
        @pl.when(t > 0)
        def _(didx=didx, vals=vals, ssem=ssem):
          for d in drains(didx, vals, ssem, KCH):
            d.wait()
        process(base + half * KCH, KCH, half * KCH, didx, vals, ssem)
      return carry

    lax.fori_loop(0, bpt // (2 * KCH), pair, None)
    for d in drains(*sets[0], KCH):
      d.wait()

    if tail_units:
      u, first, max_u = _tail_geometry(wid, tail_units)
      for uu in range(1, max_u + 1):
        @pl.when(u == uu)
        def _(uu=uu):
          nr = 8 * uu
          process(NW * bpt + 8 * first, nr, 0, *sets[0])
          for d in drains(*sets[0], nr):
            d.wait()

    for d in drains(*sets[1], KCH):
      d.wait()
    plsc.subcore_barrier()
    _write_out(bnc, acc_sh, out_h, c, s, n_sl, npad)

  return pl.kernel(
      body,
      out_type=jax.ShapeDtypeStruct((NC * npad,), jnp.float32),
      mesh=_mesh(),
      scratch_types=scratch,
      compiler_params=pltpu.CompilerParams(needs_layout_passes=False),
  )


def _tc_prep(rows):
  def body(degp, xp, dis, p):
    deg = degp[0] + degp[1] + 1.0
    d = lax.rsqrt(deg)
    d = d * (1.5 - 0.5 * deg * d * d)
    d = d * (1.5 - 0.5 * deg * d * d)
    dis[...] = d
    p[...] = d * xp[...]

  return pl.pallas_call(
      body,
      out_shape=[jax.ShapeDtypeStruct((rows, 128), jnp.float32)] * 2,
  )


def _tc_act(rows, width):
  def body(sp, dis, xp, w1, b1, w2, q):
    d = dis[...]
    s1 = d * (sp[0] + sp[1] + d * xp[...])
    acc = jnp.zeros_like(s1)
    for k in range(width):
      r = jnp.maximum(s1 * w1[k] + b1[k], 0.0)
      r = r.astype(jnp.bfloat16).astype(jnp.float32)
      w2k = w2[k].astype(jnp.bfloat16).astype(jnp.float32)
      acc = acc + r * w2k
    q[...] = d * acc

  smem = pl.BlockSpec(memory_space=pltpu.SMEM)
  return pl.pallas_call(
      body,
      in_specs=[pl.BlockSpec((NC, rows, 128), lambda: (0, 0, 0)),
                pl.BlockSpec((rows, 128), lambda: (0, 0)),
                pl.BlockSpec((rows, 128), lambda: (0, 0)),
                smem, smem, smem],
      out_shape=jax.ShapeDtypeStruct((rows, 128), jnp.float32),
  )


def _tc_fin(rows, n):
  def body(sp, dis, q, b2, out):
    d = dis[...]
    v = d * (sp[0] + sp[1]) + d * q[...] + b2[0]
    rid = lax.broadcasted_iota(jnp.int32, (rows, 128), 0)
    cid = lax.broadcasted_iota(jnp.int32, (rows, 128), 1)
    v = jnp.where(rid * 128 + cid < n, v, -jnp.inf)
    out[...] = jnp.max(v).reshape(1, 1)

  smem = pl.BlockSpec(memory_space=pltpu.SMEM)
  return pl.pallas_call(
      body,
      in_specs=[pl.BlockSpec((NC, rows, 128), lambda: (0, 0, 0)),
                pl.BlockSpec((rows, 128), lambda: (0, 0)),
                pl.BlockSpec((rows, 128), lambda: (0, 0)),
                smem],
      out_shape=jax.ShapeDtypeStruct((1, 1), jnp.float32),
  )


def kernel(x, edge_index, W1, b1, W2, b2):
  n = x.shape[0]
  e = edge_index.shape[1]
  width = W1.shape[1]

  n_sl = -(-(n + 1) // (NS * 8)) * 8
  npad = NS * n_sl
  rows = npad // 128
  nrows = e // LB

  ei3 = edge_index.reshape(2, nrows, LB)
  xp = jnp.pad(x[:, 0], (0, npad - n)).reshape(rows, 128)

  conv = _conv_pass(nrows, npad, n_sl)

  degp = _deg_pass(nrows, npad, n_sl)(ei3)
  dis, p = _tc_prep(rows)(degp.reshape(NC, rows, 128), xp)
  sp1 = conv(ei3, p.reshape(npad))
  q = _tc_act(rows, width)(sp1.reshape(NC, rows, 128), dis, xp,
                           W1.reshape(width), b1, W2.reshape(width))
  sp2 = conv(ei3, q.reshape(npad))
  return _tc_fin(rows, n)(sp2.reshape(NC, rows, 128), dis, q, b2)
